# baseline (speedup 1.0000x reference)
import jax
import jax.numpy as jnp
import numpy as np
from jax.experimental import pallas as pl
from jax.experimental.pallas import tpu as pltpu

B = 128
V = 100000
BLK = 2048
NBLK = (V + BLK - 1) // BLK
TAIL_N = V - (NBLK - 1) * BLK
_NEG_INF = np.float32(-np.inf)

_K0 = np.uint32(0)
_K1 = np.uint32(42)
_K2 = np.uint32(_K0 ^ _K1 ^ np.uint32(0x1BD11BDA))
_KS = (_K0, _K1, _K2)
_ROT0 = (13, 15, 26, 6)
_ROT1 = (17, 29, 16, 24)
_TINY = np.float32(np.finfo(np.float32).tiny)


def _rotl(x, d):
    return (x << np.uint32(d)) | (x >> np.uint32(32 - d))


def _gumbel_bits(x1):
    x0 = jnp.full_like(x1, _KS[0])
    x1 = x1 + _KS[1]
    schedule = (
        (_ROT0, _KS[1], _KS[2], 1),
        (_ROT1, _KS[2], _KS[0], 2),
        (_ROT0, _KS[0], _KS[1], 3),
        (_ROT1, _KS[1], _KS[2], 4),
        (_ROT0, _KS[2], _KS[0], 5),
    )
    for rots, ka, kb, c in schedule:
        for d in rots:
            x0 = x0 + x1
            x1 = _rotl(x1, d)
            x1 = x1 ^ x0
        x0 = x0 + ka
        x1 = x1 + (kb + np.uint32(c))
    bits = x0 ^ x1
    fl = jax.lax.bitcast_convert_type(
        (bits >> np.uint32(9)) | np.uint32(0x3F800000), jnp.float32
    ) - np.float32(1.0)
    return -jnp.log(-jnp.log(fl + _TINY))


def _scan_kernel(x_ref, a_ref, tail_ref, lp_ref, act_ref, m_ref,
                 s_ref, zb_ref, ib_ref, ab_ref):
    j = pl.program_id(0)

    off = pl.multiple_of(jnp.minimum(j, NBLK - 2) * BLK, BLK)
    xj = x_ref[:, pl.ds(off, BLK)]
    xt = tail_ref[...]
    x = jnp.where(j == NBLK - 1, xt, xj)
    a_blk = a_ref[0]

    c0 = jnp.where(j == NBLK - 1, V - BLK, j * BLK)
    gcol = jax.lax.broadcasted_iota(jnp.int32, (B, BLK), 1) + c0
    row = jax.lax.broadcasted_iota(jnp.int32, (B, BLK), 0)
    g = _gumbel_bits((row * V + gcol).astype(jnp.uint32))

    z = x + g

    bmax = jnp.max(x, axis=1, keepdims=True)
    bz = jnp.max(z, axis=1, keepdims=True)
    bidx = jnp.min(jnp.where(z == bz, gcol, V), axis=1, keepdims=True)
    a_at = jnp.sum(jnp.where(gcol == bidx, a_blk, 0), axis=1, keepdims=True)

    @pl.when(j == 0)
    def _init():
        m_ref[...] = bmax
        s_ref[...] = jnp.sum(jnp.exp(x - bmax), axis=1, keepdims=True)
        zb_ref[...] = bz
        ib_ref[...] = bidx
        ab_ref[...] = a_at

    @pl.when(j > 0)
    def _update():
        m_old = m_ref[...]
        m_new = jnp.maximum(m_old, bmax)
        bsum = jnp.sum(jnp.exp(x - m_new), axis=1, keepdims=True)
        s_ref[...] = s_ref[...] * jnp.exp(m_old - m_new) + bsum
        m_ref[...] = m_new
        upd = bz > zb_ref[...]
        zb_ref[...] = jnp.where(upd, bz, zb_ref[...])
        ib_ref[...] = jnp.where(upd, bidx, ib_ref[...])
        ab_ref[...] = jnp.where(upd, a_at, ab_ref[...])

    @pl.when(j == NBLK - 1)
    def _fin():
        rows1 = jax.lax.broadcasted_iota(jnp.int32, (B, 1), 0)
        g_best = _gumbel_bits((rows1 * V + ib_ref[...]).astype(jnp.uint32))
        lp_ref[...] = (zb_ref[...] - g_best) - (
            m_ref[...] + jnp.log(s_ref[...]))
        act_ref[...] = ab_ref[...]


@jax.jit
def kernel(logits, action_space):
    a_arr = jnp.concatenate(
        [action_space[:(NBLK - 1) * BLK], action_space[V - BLK:]]
    ).reshape(NBLK, 1, BLK)
    tail = jnp.concatenate(
        [jnp.full((B, BLK - TAIL_N), _NEG_INF, jnp.float32),
         jax.lax.slice(logits, (0, (NBLK - 1) * BLK), (B, V))], axis=1)
    lp, act = pl.pallas_call(
        _scan_kernel,
        grid=(NBLK,),
        in_specs=[
            pl.BlockSpec((B, V), lambda j: (0, 0)),
            pl.BlockSpec((1, 1, BLK), lambda j: (j, 0, 0)),
            pl.BlockSpec((B, BLK), lambda j: (0, 0)),
        ],
        out_specs=[
            pl.BlockSpec((B, 1), lambda j: (0, 0)),
            pl.BlockSpec((B, 1), lambda j: (0, 0)),
        ],
        out_shape=[
            jax.ShapeDtypeStruct((B, 1), jnp.float32),
            jax.ShapeDtypeStruct((B, 1), jnp.int32),
        ],
        scratch_shapes=[
            pltpu.VMEM((B, 1), jnp.float32),
            pltpu.VMEM((B, 1), jnp.float32),
            pltpu.VMEM((B, 1), jnp.float32),
            pltpu.VMEM((B, 1), jnp.int32),
            pltpu.VMEM((B, 1), jnp.int32),
        ],
    )(logits, a_arr, tail)
    return lp.reshape(B), act.reshape(B)


# speedup vs baseline: 1.0067x; 1.0067x over previous
import jax
import jax.numpy as jnp
import numpy as np
from jax.experimental import pallas as pl
from jax.experimental.pallas import tpu as pltpu

B = 128
V = 100000
RB = 8
CH = 2048
NCH = V // CH
UNROLL = 6
TAIL0 = NCH * CH
TAIL0_N = 1664
TAIL1 = TAIL0 + TAIL0_N
TAIL1_N = 32
_NEG_INF = np.float32(-np.inf)

_K0 = np.uint32(0)
_K1 = np.uint32(42)
_K2 = np.uint32(_K0 ^ _K1 ^ np.uint32(0x1BD11BDA))
_KS = (_K0, _K1, _K2)
_ROT0 = (13, 15, 26, 6)
_ROT1 = (17, 29, 16, 24)
_TINY = np.float32(np.finfo(np.float32).tiny)


def _rotl(x, d):
    return (x << np.uint32(d)) | (x >> np.uint32(32 - d))


def _gumbel_bits(x1):
    x0 = jnp.full_like(x1, _KS[0])
    x1 = x1 + _KS[1]
    schedule = (
        (_ROT0, _KS[1], _KS[2], 1),
        (_ROT1, _KS[2], _KS[0], 2),
        (_ROT0, _KS[0], _KS[1], 3),
        (_ROT1, _KS[1], _KS[2], 4),
        (_ROT0, _KS[2], _KS[0], 5),
    )
    for rots, ka, kb, c in schedule:
        for d in rots:
            x0 = x0 + x1
            x1 = _rotl(x1, d)
            x1 = x1 ^ x0
        x0 = x0 + ka
        x1 = x1 + (kb + np.uint32(c))
    bits = x0 ^ x1
    fl = jax.lax.bitcast_convert_type(
        (bits >> np.uint32(9)) | np.uint32(0x3F800000), jnp.float32
    ) - np.float32(1.0)
    return -jnp.log(-jnp.log(fl + _TINY))


def _scan_kernel(x_ref, a_ref, lp_ref, act_ref):
    i = pl.program_id(0)
    row_base = i * RB

    def chunk_stats(c0, n):
        x = x_ref[:, pl.ds(c0, n)]
        a_blk = a_ref[:, pl.ds(c0, n)]
        gcol = jax.lax.broadcasted_iota(jnp.int32, (RB, n), 1) + c0
        row = jax.lax.broadcasted_iota(jnp.int32, (RB, n), 0) + row_base
        counter = (row * V + gcol).astype(jnp.uint32)
        z = x + _gumbel_bits(counter)
        bmax = jnp.max(x, axis=1, keepdims=True)
        bz = jnp.max(z, axis=1, keepdims=True)
        bidx = jnp.min(jnp.where(z == bz, gcol, V), axis=1, keepdims=True)
        a_at = jnp.sum(jnp.where(gcol == bidx, a_blk, 0), axis=1,
                       keepdims=True)
        bsum = jnp.sum(jnp.exp(x - bmax), axis=1, keepdims=True)
        return bmax, bz, bidx, a_at, bsum

    def merge(carry, stats):
        m, s, zb, ib, ab = carry
        bmax, bz, bidx, a_at, bsum = stats
        m_new = jnp.maximum(m, bmax)
        s = s * jnp.exp(m - m_new) + bsum * jnp.exp(bmax - m_new)
        upd = bz > zb
        zb = jnp.where(upd, bz, zb)
        ib = jnp.where(upd, bidx, ib)
        ab = jnp.where(upd, a_at, ab)
        return m_new, s, zb, ib, ab

    def body(c, carry):
        c0 = pl.multiple_of(c * CH, CH)
        return merge(carry, chunk_stats(c0, CH))

    neg = jnp.full((RB, 1), _NEG_INF, jnp.float32)
    zero = jnp.zeros((RB, 1), jnp.float32)
    izero = jnp.zeros((RB, 1), jnp.int32)
    carry = (neg, zero, neg, izero, izero)
    carry = jax.lax.fori_loop(0, NCH, body, carry, unroll=UNROLL)
    carry = merge(carry, chunk_stats(TAIL0, TAIL0_N))
    m, s, zb, ib, ab = merge(carry, chunk_stats(TAIL1, TAIL1_N))

    rows1 = jax.lax.broadcasted_iota(jnp.int32, (RB, 1), 0) + row_base
    g_best = _gumbel_bits((rows1 * V + ib).astype(jnp.uint32))
    lp_ref[...] = (zb - g_best) - (m + jnp.log(s))
    act_ref[...] = ab


@jax.jit
def kernel(logits, action_space):
    a2d = action_space.reshape(1, V)
    lp, act = pl.pallas_call(
        _scan_kernel,
        grid=(B // RB,),
        in_specs=[
            pl.BlockSpec((RB, V), lambda i: (i, 0)),
            pl.BlockSpec((1, V), lambda i: (0, 0)),
        ],
        out_specs=[
            pl.BlockSpec((RB, 1), lambda i: (i, 0)),
            pl.BlockSpec((RB, 1), lambda i: (i, 0)),
        ],
        out_shape=[
            jax.ShapeDtypeStruct((B, 1), jnp.float32),
            jax.ShapeDtypeStruct((B, 1), jnp.int32),
        ],
    )(logits, a2d)
    return lp.reshape(B), act.reshape(B)


# speedup vs baseline: 1.0452x; 1.0382x over previous
import jax
import jax.numpy as jnp
import numpy as np
from jax.experimental import pallas as pl
from jax.experimental.pallas import tpu as pltpu

B = 128
V = 100000
BLK = 2048
NBLK = (V + BLK - 1) // BLK
TAIL_N = V - (NBLK - 1) * BLK
_NEG_INF = np.float32(-np.inf)

_K0 = np.uint32(0)
_K1 = np.uint32(42)
_K2 = np.uint32(_K0 ^ _K1 ^ np.uint32(0x1BD11BDA))
_KS = (_K0, _K1, _K2)
_ROT0 = (13, 15, 26, 6)
_ROT1 = (17, 29, 16, 24)
_TINY = np.float32(np.finfo(np.float32).tiny)


def _rotl(x, d):
    return (x << np.uint32(d)) | (x >> np.uint32(32 - d))


def _gumbel_bits(x1):
    x0 = jnp.full_like(x1, _KS[0])
    x1 = x1 + _KS[1]
    schedule = (
        (_ROT0, _KS[1], _KS[2], 1),
        (_ROT1, _KS[2], _KS[0], 2),
        (_ROT0, _KS[0], _KS[1], 3),
        (_ROT1, _KS[1], _KS[2], 4),
        (_ROT0, _KS[2], _KS[0], 5),
    )
    for rots, ka, kb, c in schedule:
        for d in rots:
            x0 = x0 + x1
            x1 = _rotl(x1, d)
            x1 = x1 ^ x0
        x0 = x0 + ka
        x1 = x1 + (kb + np.uint32(c))
    bits = x0 ^ x1
    fl = jax.lax.bitcast_convert_type(
        (bits >> np.uint32(9)) | np.uint32(0x3F800000), jnp.float32
    ) - np.float32(1.0)
    return -jnp.log(-jnp.log(fl + _TINY))


def _hbm_copy(x_hbm, buf, sem, jj):
    slot = jax.lax.rem(jj, 2)
    return pltpu.make_async_copy(
        x_hbm.at[:, pl.ds(jj * BLK, BLK)], buf.at[slot], sem.at[slot])


def _scan_kernel(x_hbm, a_ref, tail_ref, lp_ref, act_ref, buf, sem, m_ref,
                 s_ref, zb_ref, ib_ref, ab_ref):
    j = pl.program_id(0)

    @pl.when(j == 0)
    def _prime():
        _hbm_copy(x_hbm, buf, sem, j).start()

    @pl.when(j < NBLK - 2)
    def _ahead():
        _hbm_copy(x_hbm, buf, sem, j + 1).start()

    @pl.when(j == NBLK - 2)
    def _ahead_tail():
        pltpu.make_async_copy(tail_ref, buf.at[0], sem.at[0]).start()

    @pl.when(j < NBLK - 1)
    def _wait_main():
        _hbm_copy(x_hbm, buf, sem, j).wait()

    @pl.when(j == NBLK - 1)
    def _wait_tail():
        pltpu.make_async_copy(tail_ref, buf.at[0], sem.at[0]).wait()

    x = buf[jax.lax.rem(j, 2)]
    a_blk = a_ref[0]

    c0 = jnp.where(j == NBLK - 1, V - BLK, j * BLK)
    gcol = jax.lax.broadcasted_iota(jnp.int32, (B, BLK), 1) + c0
    row = jax.lax.broadcasted_iota(jnp.int32, (B, BLK), 0)
    g = _gumbel_bits((row * V + gcol).astype(jnp.uint32))

    z = x + g

    bmax = jnp.max(x, axis=1, keepdims=True)
    bz = jnp.max(z, axis=1, keepdims=True)
    bidx = jnp.min(jnp.where(z == bz, gcol, V), axis=1, keepdims=True)
    a_at = jnp.sum(jnp.where(gcol == bidx, a_blk, 0), axis=1, keepdims=True)

    @pl.when(j == 0)
    def _init():
        m_ref[...] = bmax
        s_ref[...] = jnp.sum(jnp.exp(x - bmax), axis=1, keepdims=True)
        zb_ref[...] = bz
        ib_ref[...] = bidx
        ab_ref[...] = a_at

    @pl.when(j > 0)
    def _update():
        m_old = m_ref[...]
        m_new = jnp.maximum(m_old, bmax)
        bsum = jnp.sum(jnp.exp(x - m_new), axis=1, keepdims=True)
        s_ref[...] = s_ref[...] * jnp.exp(m_old - m_new) + bsum
        m_ref[...] = m_new
        upd = bz > zb_ref[...]
        zb_ref[...] = jnp.where(upd, bz, zb_ref[...])
        ib_ref[...] = jnp.where(upd, bidx, ib_ref[...])
        ab_ref[...] = jnp.where(upd, a_at, ab_ref[...])

    @pl.when(j == NBLK - 1)
    def _fin():
        rows1 = jax.lax.broadcasted_iota(jnp.int32, (B, 1), 0)
        g_best = _gumbel_bits((rows1 * V + ib_ref[...]).astype(jnp.uint32))
        lp_ref[...] = (zb_ref[...] - g_best) - (
            m_ref[...] + jnp.log(s_ref[...]))
        act_ref[...] = ab_ref[...]


@jax.jit
def kernel(logits, action_space):
    a_arr = jnp.concatenate(
        [action_space[:(NBLK - 1) * BLK], action_space[V - BLK:]]
    ).reshape(NBLK, 1, BLK)
    tail = jnp.concatenate(
        [jnp.full((B, BLK - TAIL_N), _NEG_INF, jnp.float32),
         jax.lax.slice(logits, (0, (NBLK - 1) * BLK), (B, V))], axis=1)
    lp, act = pl.pallas_call(
        _scan_kernel,
        grid=(NBLK,),
        in_specs=[
            pl.BlockSpec(memory_space=pl.ANY),
            pl.BlockSpec((1, 1, BLK), lambda j: (j, 0, 0)),
            pl.BlockSpec((B, BLK), lambda j: (0, 0)),
        ],
        out_specs=[
            pl.BlockSpec((B, 1), lambda j: (0, 0)),
            pl.BlockSpec((B, 1), lambda j: (0, 0)),
        ],
        out_shape=[
            jax.ShapeDtypeStruct((B, 1), jnp.float32),
            jax.ShapeDtypeStruct((B, 1), jnp.int32),
        ],
        scratch_shapes=[
            pltpu.VMEM((2, B, BLK), jnp.float32),
            pltpu.SemaphoreType.DMA((2,)),
            pltpu.VMEM((B, 1), jnp.float32),
            pltpu.VMEM((B, 1), jnp.float32),
            pltpu.VMEM((B, 1), jnp.float32),
            pltpu.VMEM((B, 1), jnp.int32),
            pltpu.VMEM((B, 1), jnp.int32),
        ],
    )(logits, a_arr, tail)
    return lp.reshape(B), act.reshape(B)
